# baseline (speedup 1.0000x reference)
import functools

import jax
import jax.numpy as jnp
from jax import lax
from jax.experimental import pallas as pl
from jax.experimental.pallas import tpu as pltpu
from jax.experimental.pallas import tpu_sc as plsc

_B = 4096
_N = 1000
_LANES = 16
_CHUNK = 16
_NFULL = _N // _LANES
_TAIL_OFF = _N - _LANES


def _remap_body(nc, rows_per_w, x_hbm, map_hbm, out_hbm,
                map_v, in_buf, out_buf,
                sem_in0, sem_in1, sem_out0, sem_out1):
    cid = lax.axis_index("c")
    sid = lax.axis_index("s")
    wid = sid * nc + cid
    base0 = wid * rows_per_w
    nchunks = rows_per_w // _CHUNK

    pltpu.sync_copy(map_hbm, map_v)

    sem_in = (sem_in0, sem_in1)
    sem_out = (sem_out0, sem_out1)

    def start_in(g):
        return pltpu.async_copy(
            x_hbm.at[pl.ds(base0 + g * _CHUNK, _CHUNK)],
            in_buf.at[g % 2], sem_in[g % 2])

    def start_out(g):
        return pltpu.async_copy(
            out_buf.at[g % 2],
            out_hbm.at[pl.ds(base0 + g * _CHUNK, _CHUNK)], sem_out[g % 2])

    row_ids = [jnp.full((_LANES,), r, jnp.int32) for r in range(_CHUNK)]

    def block(ph, col_off, idxk):
        idxc = jnp.minimum(idxk, _N - 1)
        ok = idxk < _N
        for r in range(_CHUNK):
            vals = plsc.load_gather(in_buf.at[ph], [row_ids[r], idxc])
            out_buf[ph, r, pl.ds(col_off, _LANES)] = jnp.where(ok, vals, 0.0)

    def compute_chunk(ph):
        @plsc.parallel_loop(0, _NFULL, unroll=4)
        def _(k):
            block(ph, k * _LANES, map_v[pl.ds(k * _LANES, _LANES)])
        block(ph, _TAIL_OFF, map_v[pl.ds(_TAIL_OFF, _LANES)])

    pending_in = {0: start_in(0)}
    pending_out = {}

    for g in range(nchunks):
        ph = g % 2
        if g + 1 < nchunks:
            pending_in[g + 1] = start_in(g + 1)
        pending_in.pop(g).wait()
        if g - 2 in pending_out:
            pending_out.pop(g - 2).wait()
        compute_chunk(ph)
        pending_out[g] = start_out(g)

    for g in sorted(pending_out):
        pending_out.pop(g).wait()


def kernel(x, mapping):
    mapping = mapping.astype(jnp.int32)

    info = plsc.get_sparse_core_info()
    nw = info.num_cores * info.num_subcores
    rows_per_w = _B // nw

    mesh = plsc.VectorSubcoreMesh(core_axis_name="c", subcore_axis_name="s")
    f = pl.kernel(
        functools.partial(_remap_body, info.num_cores, rows_per_w),
        out_type=jax.ShapeDtypeStruct((_B, _N), jnp.float32),
        mesh=mesh,
        compiler_params=pltpu.CompilerParams(
            needs_layout_passes=False, use_tc_tiling_on_sc=True),
        scratch_types=[
            pltpu.VMEM((_N,), jnp.int32),
            pltpu.VMEM((2, _CHUNK, _N), jnp.float32),
            pltpu.VMEM((2, _CHUNK, _N), jnp.float32),
            pltpu.SemaphoreType.DMA,
            pltpu.SemaphoreType.DMA,
            pltpu.SemaphoreType.DMA,
            pltpu.SemaphoreType.DMA,
        ],
    )
    return f(x, mapping)


# speedup vs baseline: 2.1452x; 2.1452x over previous
import functools

import jax
import jax.numpy as jnp
from jax import lax
from jax.experimental import pallas as pl
from jax.experimental.pallas import tpu as pltpu
from jax.experimental.pallas import tpu_sc as plsc

_B = 4096
_N = 1000
_LANES = 16
_ROWS_PER_W = 32
_CHUNK = 8
_NCHUNKS = _ROWS_PER_W // _CHUNK


def _remap_body(nc, xt_hbm, map_hbm, out_hbm,
                idx_raw, idx_c, rows_v,
                sem_in0, sem_in1, sem_out0, sem_out1):
    cid = lax.axis_index("c")
    sid = lax.axis_index("s")
    wid = sid * nc + cid
    base = jnp.minimum(wid * _ROWS_PER_W, _N - _ROWS_PER_W)

    pltpu.sync_copy(map_hbm.at[pl.ds(base, _ROWS_PER_W)], idx_raw)
    for v in range(_ROWS_PER_W // _LANES):
        idx_c[pl.ds(v * _LANES, _LANES)] = jnp.minimum(
            idx_raw[pl.ds(v * _LANES, _LANES)], _N - 1)

    sem_in = (sem_in0, sem_in1)
    sem_out = (sem_out0, sem_out1)
    zeros = jnp.zeros((_LANES,), jnp.float32)

    pending_out = {}
    for c in range(_NCHUNKS):
        ph = c % 2
        if c - 2 in pending_out:
            pending_out.pop(c - 2).wait()
        pltpu.async_copy(
            xt_hbm.at[idx_c.at[pl.ds(c * _CHUNK, _CHUNK)]],
            rows_v.at[ph], sem_in[ph]).wait()
        for r in range(_CHUNK):
            bvec = plsc.load_gather(
                idx_raw, [jnp.full((_LANES,), c * _CHUNK + r, jnp.int32)])
            sentinel = lax.reduce_max(bvec, (0,))

            @pl.when(sentinel >= _N)
            def _():
                def zstep(i, _):
                    rows_v[ph, r, pl.ds(i * _LANES, _LANES)] = zeros
                    return 0
                lax.fori_loop(0, _B // _LANES, zstep, 0)
        pending_out[c] = pltpu.async_copy(
            rows_v.at[ph],
            out_hbm.at[pl.ds(base + c * _CHUNK, _CHUNK)], sem_out[ph])

    for c in sorted(pending_out):
        pending_out.pop(c).wait()


def kernel(x, mapping):
    mapping = mapping.astype(jnp.int32)

    info = plsc.get_sparse_core_info()
    nw = info.num_cores * info.num_subcores
    assert nw * _ROWS_PER_W >= _N

    mesh = plsc.VectorSubcoreMesh(core_axis_name="c", subcore_axis_name="s")
    f = pl.kernel(
        functools.partial(_remap_body, info.num_cores),
        out_type=jax.ShapeDtypeStruct((_N, _B), jnp.float32),
        mesh=mesh,
        compiler_params=pltpu.CompilerParams(needs_layout_passes=False),
        scratch_types=[
            pltpu.VMEM((_ROWS_PER_W,), jnp.int32),
            pltpu.VMEM((_ROWS_PER_W,), jnp.int32),
            pltpu.VMEM((2, _CHUNK, _B), jnp.float32),
            pltpu.SemaphoreType.DMA,
            pltpu.SemaphoreType.DMA,
            pltpu.SemaphoreType.DMA,
            pltpu.SemaphoreType.DMA,
        ],
    )
    return f(x.T, mapping).T


# speedup vs baseline: 2.2401x; 1.0442x over previous
import functools

import jax
import jax.numpy as jnp
from jax import lax
from jax.experimental import pallas as pl
from jax.experimental.pallas import tpu as pltpu
from jax.experimental.pallas import tpu_sc as plsc

_B = 4096
_N = 1000
_LANES = 16
_ROWS_PER_W = 32
_CHUNK = 8
_NCHUNKS = _ROWS_PER_W // _CHUNK


def _remap_body(nc, xt_hbm, map_hbm, out_hbm,
                idx_raw, idx_c, rows_v,
                sem_in0, sem_in1, sem_out0, sem_out1):
    cid = lax.axis_index("c")
    sid = lax.axis_index("s")
    wid = sid * nc + cid
    base = jnp.minimum(wid * _ROWS_PER_W, _N - _ROWS_PER_W)

    pltpu.sync_copy(map_hbm.at[pl.ds(base, _ROWS_PER_W)], idx_raw)
    for v in range(_ROWS_PER_W // _LANES):
        idx_c[pl.ds(v * _LANES, _LANES)] = jnp.minimum(
            idx_raw[pl.ds(v * _LANES, _LANES)], _N - 1)

    sem_in = (sem_in0, sem_in1)
    sem_out = (sem_out0, sem_out1)
    zeros = jnp.zeros((_LANES,), jnp.float32)

    def start_in(c):
        ph = c % 3
        return pltpu.async_copy(
            xt_hbm.at[idx_c.at[pl.ds(c * _CHUNK, _CHUNK)]],
            rows_v.at[ph], sem_in[c % 2])

    pending_in = {0: start_in(0), 1: start_in(1)}
    pending_out = {}
    for c in range(_NCHUNKS):
        ph = c % 3
        pending_in.pop(c).wait()
        for r in range(_CHUNK):
            bvec = plsc.load_gather(
                idx_raw, [jnp.full((_LANES,), c * _CHUNK + r, jnp.int32)])
            sentinel = lax.reduce_max(bvec, (0,))

            @pl.when(sentinel >= _N)
            def _():
                def zstep(i, _):
                    rows_v[ph, r, pl.ds(i * _LANES, _LANES)] = zeros
                    return 0
                lax.fori_loop(0, _B // _LANES, zstep, 0)
        pending_out[c] = pltpu.async_copy(
            rows_v.at[ph],
            out_hbm.at[pl.ds(base + c * _CHUNK, _CHUNK)], sem_out[c % 2])
        if c + 2 < _NCHUNKS:
            if c - 1 in pending_out:
                pending_out.pop(c - 1).wait()
            pending_in[c + 2] = start_in(c + 2)

    for c in sorted(pending_out):
        pending_out.pop(c).wait()


def kernel(x, mapping):
    mapping = mapping.astype(jnp.int32)

    info = plsc.get_sparse_core_info()
    nw = info.num_cores * info.num_subcores
    assert nw * _ROWS_PER_W >= _N

    mesh = plsc.VectorSubcoreMesh(core_axis_name="c", subcore_axis_name="s")
    f = pl.kernel(
        functools.partial(_remap_body, info.num_cores),
        out_type=jax.ShapeDtypeStruct((_N, _B), jnp.float32),
        mesh=mesh,
        compiler_params=pltpu.CompilerParams(needs_layout_passes=False),
        scratch_types=[
            pltpu.VMEM((_ROWS_PER_W,), jnp.int32),
            pltpu.VMEM((_ROWS_PER_W,), jnp.int32),
            pltpu.VMEM((3, _CHUNK, _B), jnp.float32),
            pltpu.SemaphoreType.DMA,
            pltpu.SemaphoreType.DMA,
            pltpu.SemaphoreType.DMA,
            pltpu.SemaphoreType.DMA,
        ],
    )
    return f(x.T, mapping).T
